# baseline (speedup 1.0000x reference)
import functools

import jax
import jax.numpy as jnp
from jax import lax
from jax.experimental import pallas as pl
from jax.experimental.pallas import tpu as pltpu
from jax.experimental.pallas import tpu_sc as plsc

B = 16384
F = 64
NC = 2
NS = 16
NW = NC * NS
L = 16
BLK = 128
DEPTH = 8
CHUNK = 64

_i32 = jnp.int32


def _band(a, b):
    return lax.bitwise_and(a, _i32(b))


def _gather_kernel(sidx, slotinfo, dlist, tbl_t, n_total):
    mesh = plsc.VectorSubcoreMesh(core_axis_name="c", subcore_axis_name="s")
    cp = pltpu.CompilerParams(
        needs_layout_passes=False,
        use_tc_tiling_on_sc=True,
        disable_bounds_checks=True,
    )
    n = n_total // NW
    out_type = jax.ShapeDtypeStruct((n_total * F,), jnp.float32)

    @functools.partial(
        pl.kernel,
        out_type=out_type,
        mesh=mesh,
        compiler_params=cp,
        scratch_types=[
            pltpu.VMEM((n,), _i32),
            pltpu.VMEM((n,), _i32),
            pltpu.VMEM((n + 16,), _i32),
            pltpu.VMEM((DEPTH, F, BLK), jnp.float32),
            pltpu.VMEM((2, CHUNK * F), jnp.float32),
            pltpu.SemaphoreType.DMA,
            pltpu.SemaphoreType.DMA,
        ],
    )
    def k(sidx_hbm, slotinfo_hbm, dlist_hbm, tbl, gout,
          sidx_v, slot_v, dl_v, arena, staging, sem, sem_out):
        wid = lax.axis_index("s") * NC + lax.axis_index("c")
        lane = lax.iota(_i32, L)
        zero16 = jnp.zeros((L,), _i32)
        base = wid * n

        def extract(vec_ref, r):
            off = pl.multiple_of((r // L) * L, L)
            chunk = vec_ref[pl.ds(off, L)]
            return jnp.sum(jnp.where(lane == (r % L), chunk, zero16))

        pltpu.sync_copy(sidx_hbm.at[pl.ds(base, n)], sidx_v)
        pltpu.sync_copy(slotinfo_hbm.at[pl.ds(base, n)], slot_v)
        s0 = lax.shift_right_logical(extract(slot_v, 0), 1)
        slast = lax.shift_right_logical(extract(slot_v, n - 1), 1)
        s0a = pl.multiple_of(
            lax.shift_left(lax.shift_right_logical(s0, 3), 3), 8)
        pltpu.sync_copy(dlist_hbm.at[pl.ds(s0a, n + 8)],
                        dl_v.at[pl.ds(0, n + 8)])

        def issue(s):
            b = extract(dl_v, s - s0a)
            off = pl.multiple_of(b * BLK, BLK)
            pltpu.async_copy(tbl.at[:, pl.ds(off, BLK)],
                             arena.at[_band(s, DEPTH - 1)], sem)

        for kk in range(DEPTH - 1):
            @pl.when(s0 + kk <= slast)
            def _():
                issue(s0 + kk)

        @pl.loop(0, n)
        def _(r):
            rr = _band(r, CHUNK - 1)
            ck = lax.shift_right_logical(r, 6)
            p = _band(ck, 1)

            @pl.when(jnp.logical_and(rr == 0, ck >= 2))
            def _():
                pltpu.make_async_copy(
                    gout.at[pl.ds(0, CHUNK * F)], staging.at[p],
                    sem_out).wait()

            info = extract(slot_v, r)
            s = lax.shift_right_logical(info, 1)
            isn = _band(info, 1)
            d = _band(s, DEPTH - 1)

            @pl.when(jnp.logical_or(isn == 1, r == 0))
            def _():
                pltpu.make_async_copy(
                    tbl.at[:, pl.ds(0, BLK)], arena.at[d], sem).wait()
                s2 = s + DEPTH - 1

                @pl.when(s2 <= slast)
                def _():
                    issue(s2)

            c = _band(extract(sidx_v, r), BLK - 1)
            csp = jnp.full((L,), c, _i32)
            dsp = jnp.full((L,), d, _i32)
            for g in range(4):
                v = plsc.load_gather(arena, [dsp, lane + g * L, csp])
                soff = pl.multiple_of(rr * F + g * L, L)
                staging[p, pl.ds(soff, L)] = v

            @pl.when(rr == CHUNK - 1)
            def _():
                goff = pl.multiple_of(
                    (base + r - (CHUNK - 1)) * F, CHUNK * F)
                pltpu.async_copy(
                    staging.at[p], gout.at[pl.ds(goff, CHUNK * F)],
                    sem_out)

        for _unused in range(2):
            pltpu.make_async_copy(
                gout.at[pl.ds(0, CHUNK * F)], staging.at[0],
                sem_out).wait()

    return k(sidx, slotinfo, dlist, tbl_t)


def _dot_kernel(rk_u, rk_i, rk_j, gu, gi):
    mesh = plsc.VectorSubcoreMesh(core_axis_name="c", subcore_axis_name="s")
    cp = pltpu.CompilerParams(
        needs_layout_passes=False, use_tc_tiling_on_sc=False)
    BPW = B // NW
    out_type = (
        jax.ShapeDtypeStruct((B,), jnp.float32),
        jax.ShapeDtypeStruct((B,), jnp.float32),
    )

    @functools.partial(
        pl.kernel,
        out_type=out_type,
        mesh=mesh,
        compiler_params=cp,
        scratch_types=[
            pltpu.VMEM((BPW,), _i32),
            pltpu.VMEM((BPW,), _i32),
            pltpu.VMEM((BPW,), _i32),
            pltpu.VMEM((BPW, F), jnp.float32),
            pltpu.VMEM((BPW, F), jnp.float32),
            pltpu.VMEM((BPW, F), jnp.float32),
            pltpu.VMEM((BPW,), jnp.float32),
            pltpu.VMEM((BPW,), jnp.float32),
            pltpu.SemaphoreType.DMA,
            pltpu.SemaphoreType.DMA,
            pltpu.SemaphoreType.DMA,
        ],
    )
    def k(rku_hbm, rki_hbm, rkj_hbm, gu_hbm, gi_hbm, oi_hbm, oj_hbm,
          uidx, iidx, jidx, urows, irows, jrows, oi_v, oj_v, su_, si_, sj_):
        wid = lax.axis_index("s") * NC + lax.axis_index("c")
        base = wid * BPW

        pltpu.sync_copy(rku_hbm.at[pl.ds(base, BPW)], uidx)
        pltpu.sync_copy(rki_hbm.at[pl.ds(base, BPW)], iidx)
        pltpu.sync_copy(rkj_hbm.at[pl.ds(base, BPW)], jidx)

        cu = pltpu.async_copy(gu_hbm.at[uidx], urows, su_)
        ci = pltpu.async_copy(gi_hbm.at[iidx], irows, si_)
        cj = pltpu.async_copy(gi_hbm.at[jidx], jrows, sj_)
        cu.wait()
        ci.wait()
        cj.wait()

        lane = lax.iota(_i32, L)
        m15 = lane == (L - 1)

        @pl.loop(0, BPW)
        def _(r):
            u0 = urows[r, pl.ds(0, L)]
            u1 = urows[r, pl.ds(L, L)]
            u2 = urows[r, pl.ds(2 * L, L)]
            u3 = urows[r, pl.ds(3 * L, L)]
            a0 = irows[r, pl.ds(0, L)]
            a1 = irows[r, pl.ds(L, L)]
            a2 = irows[r, pl.ds(2 * L, L)]
            a3 = irows[r, pl.ds(3 * L, L)]
            b0 = jrows[r, pl.ds(0, L)]
            b1 = jrows[r, pl.ds(L, L)]
            b2 = jrows[r, pl.ds(2 * L, L)]
            b3 = jrows[r, pl.ds(3 * L, L)]
            acc_i = u0 * a0 + u1 * a1 + u2 * a2 + u3 * a3
            acc_j = u0 * b0 + u1 * b1 + u2 * b2 + u3 * b3
            rvec = jnp.full((L,), r, _i32)
            plsc.store_scatter(oi_v, [rvec], plsc.cumsum(acc_i), mask=m15)
            plsc.store_scatter(oj_v, [rvec], plsc.cumsum(acc_j), mask=m15)

        pltpu.sync_copy(oi_v, oi_hbm.at[pl.ds(base, BPW)])
        pltpu.sync_copy(oj_v, oj_hbm.at[pl.ds(base, BPW)])

    return k(rk_u, rk_i, rk_j, gu, gi)


def _sorted_stream(idx, wpos):
    n = idx.shape[0]
    iota = lax.iota(_i32, n)
    packed = lax.bitwise_or(
        lax.shift_left(lax.shift_right_logical(idx, 7), wpos), iota)
    sp = lax.sort([packed], num_keys=1)[0]
    pos = lax.bitwise_and(sp, _i32((1 << wpos) - 1))
    blocks = lax.shift_right_logical(sp, wpos)
    carr = lax.bitwise_and(jnp.take(idx, pos), _i32(127))
    rank = jnp.zeros((n,), _i32).at[pos].set(iota)
    isnew = jnp.concatenate(
        [jnp.ones((1,), _i32), (jnp.diff(blocks) != 0).astype(_i32)])
    slot = jnp.cumsum(isnew, dtype=_i32) - 1
    slotinfo = slot * 2 + isnew
    dlist = jnp.zeros((n + 16,), _i32).at[slot].set(blocks)
    return carr, slotinfo, dlist, rank


def kernel(user, item_i, item_j, embed_user, embed_item):
    user = user.astype(_i32)
    item_i = item_i.astype(_i32)
    item_j = item_j.astype(_i32)

    carr_u, slotinfo_u, dlist_u, rank_u = _sorted_stream(user, 14)
    gu_flat = _gather_kernel(carr_u, slotinfo_u, dlist_u, embed_user.T, B)

    items = jnp.concatenate([item_i, item_j])
    carr_i, slotinfo_i, dlist_i, rank_it = _sorted_stream(items, 15)
    gi_flat = _gather_kernel(carr_i, slotinfo_i, dlist_i, embed_item.T, 2 * B)

    gu = gu_flat.reshape(B, F)
    gi = gi_flat.reshape(2 * B, F)

    return _dot_kernel(rank_u, rank_it[:B], rank_it[B:], gu, gi)


# speedup vs baseline: 1.5102x; 1.5102x over previous
import functools

import jax
import jax.numpy as jnp
from jax import lax
from jax.experimental import pallas as pl
from jax.experimental.pallas import tpu as pltpu
from jax.experimental.pallas import tpu_sc as plsc

B = 16384
F = 64
NC = 2
NS = 16
NW = NC * NS
L = 16
BLK = 128
DEPTH = 8
CHUNK = 64

_i32 = jnp.int32


def _band(a, b):
    return lax.bitwise_and(a, _i32(b))


def _gather_kernel(sidx, slotinfo, dlist, tbl_t, n_total):
    mesh = plsc.VectorSubcoreMesh(core_axis_name="c", subcore_axis_name="s")
    cp = pltpu.CompilerParams(
        needs_layout_passes=False,
        use_tc_tiling_on_sc=True,
        disable_bounds_checks=True,
    )
    n = n_total // NW
    out_type = jax.ShapeDtypeStruct((n_total * F,), jnp.float32)

    @functools.partial(
        pl.kernel,
        out_type=out_type,
        mesh=mesh,
        compiler_params=cp,
        scratch_types=[
            pltpu.VMEM((n,), _i32),
            pltpu.VMEM((n,), _i32),
            pltpu.VMEM((n + 16,), _i32),
            pltpu.VMEM((DEPTH, F, BLK), jnp.float32),
            pltpu.VMEM((2, CHUNK * F), jnp.float32),
            pltpu.SemaphoreType.DMA,
            pltpu.SemaphoreType.DMA,
        ],
    )
    def k(sidx_hbm, slotinfo_hbm, dlist_hbm, tbl, gout,
          sidx_v, slot_v, dl_v, arena, staging, sem, sem_out):
        wid = lax.axis_index("s") * NC + lax.axis_index("c")
        lane = lax.iota(_i32, L)
        zero16 = jnp.zeros((L,), _i32)
        base = wid * n

        def extract(vec_ref, r):
            off = pl.multiple_of((r // L) * L, L)
            chunk = vec_ref[pl.ds(off, L)]
            return jnp.sum(jnp.where(lane == (r % L), chunk, zero16))

        pltpu.sync_copy(sidx_hbm.at[pl.ds(base, n)], sidx_v)
        pltpu.sync_copy(slotinfo_hbm.at[pl.ds(base, n)], slot_v)
        s0 = lax.shift_right_logical(extract(slot_v, 0), 1)
        slast = lax.shift_right_logical(extract(slot_v, n - 1), 1)
        s0a = pl.multiple_of(
            lax.shift_left(lax.shift_right_logical(s0, 3), 3), 8)
        pltpu.sync_copy(dlist_hbm.at[pl.ds(s0a, n + 8)],
                        dl_v.at[pl.ds(0, n + 8)])

        def issue(s):
            b = extract(dl_v, s - s0a)
            off = pl.multiple_of(b * BLK, BLK)
            pltpu.async_copy(tbl.at[:, pl.ds(off, BLK)],
                             arena.at[_band(s, DEPTH - 1)], sem)

        for kk in range(DEPTH - 1):
            @pl.when(s0 + kk <= slast)
            def _():
                issue(s0 + kk)

        @pl.loop(0, n)
        def _(r):
            rr = _band(r, CHUNK - 1)
            ck = lax.shift_right_logical(r, 6)
            p = _band(ck, 1)

            @pl.when(jnp.logical_and(rr == 0, ck >= 2))
            def _():
                pltpu.make_async_copy(
                    gout.at[pl.ds(0, CHUNK * F)], staging.at[p],
                    sem_out).wait()

            info = extract(slot_v, r)
            s = lax.shift_right_logical(info, 1)
            isn = _band(info, 1)
            d = _band(s, DEPTH - 1)

            @pl.when(jnp.logical_or(isn == 1, r == 0))
            def _():
                pltpu.make_async_copy(
                    tbl.at[:, pl.ds(0, BLK)], arena.at[d], sem).wait()
                s2 = s + DEPTH - 1

                @pl.when(s2 <= slast)
                def _():
                    issue(s2)

            c = _band(extract(sidx_v, r), BLK - 1)
            csp = jnp.full((L,), c, _i32)
            dsp = jnp.full((L,), d, _i32)
            for g in range(4):
                v = plsc.load_gather(arena, [dsp, lane + g * L, csp])
                soff = pl.multiple_of(rr * F + g * L, L)
                staging[p, pl.ds(soff, L)] = v

            @pl.when(rr == CHUNK - 1)
            def _():
                goff = pl.multiple_of(
                    (base + r - (CHUNK - 1)) * F, CHUNK * F)
                pltpu.async_copy(
                    staging.at[p], gout.at[pl.ds(goff, CHUNK * F)],
                    sem_out)

        for _unused in range(2):
            pltpu.make_async_copy(
                gout.at[pl.ds(0, CHUNK * F)], staging.at[0],
                sem_out).wait()

    return k(sidx, slotinfo, dlist, tbl_t)


def _dot_kernel(rk_u, rk_i, rk_j, gu, gi):
    mesh = plsc.VectorSubcoreMesh(core_axis_name="c", subcore_axis_name="s")
    cp = pltpu.CompilerParams(
        needs_layout_passes=False, use_tc_tiling_on_sc=False)
    BPW = B // NW
    out_type = (
        jax.ShapeDtypeStruct((B,), jnp.float32),
        jax.ShapeDtypeStruct((B,), jnp.float32),
    )

    @functools.partial(
        pl.kernel,
        out_type=out_type,
        mesh=mesh,
        compiler_params=cp,
        scratch_types=[
            pltpu.VMEM((BPW,), _i32),
            pltpu.VMEM((BPW,), _i32),
            pltpu.VMEM((BPW,), _i32),
            pltpu.VMEM((BPW, F), jnp.float32),
            pltpu.VMEM((BPW, F), jnp.float32),
            pltpu.VMEM((BPW, F), jnp.float32),
            pltpu.VMEM((BPW,), jnp.float32),
            pltpu.VMEM((BPW,), jnp.float32),
            pltpu.SemaphoreType.DMA,
            pltpu.SemaphoreType.DMA,
            pltpu.SemaphoreType.DMA,
        ],
    )
    def k(rku_hbm, rki_hbm, rkj_hbm, gu_hbm, gi_hbm, oi_hbm, oj_hbm,
          uidx, iidx, jidx, urows, irows, jrows, oi_v, oj_v, su_, si_, sj_):
        wid = lax.axis_index("s") * NC + lax.axis_index("c")
        base = wid * BPW

        pltpu.sync_copy(rku_hbm.at[pl.ds(base, BPW)], uidx)
        pltpu.sync_copy(rki_hbm.at[pl.ds(base, BPW)], iidx)
        pltpu.sync_copy(rkj_hbm.at[pl.ds(base, BPW)], jidx)

        cu = pltpu.async_copy(gu_hbm.at[uidx], urows, su_)
        ci = pltpu.async_copy(gi_hbm.at[iidx], irows, si_)
        cj = pltpu.async_copy(gi_hbm.at[jidx], jrows, sj_)
        cu.wait()
        ci.wait()
        cj.wait()

        lane = lax.iota(_i32, L)
        m15 = lane == (L - 1)

        @pl.loop(0, BPW)
        def _(r):
            u0 = urows[r, pl.ds(0, L)]
            u1 = urows[r, pl.ds(L, L)]
            u2 = urows[r, pl.ds(2 * L, L)]
            u3 = urows[r, pl.ds(3 * L, L)]
            a0 = irows[r, pl.ds(0, L)]
            a1 = irows[r, pl.ds(L, L)]
            a2 = irows[r, pl.ds(2 * L, L)]
            a3 = irows[r, pl.ds(3 * L, L)]
            b0 = jrows[r, pl.ds(0, L)]
            b1 = jrows[r, pl.ds(L, L)]
            b2 = jrows[r, pl.ds(2 * L, L)]
            b3 = jrows[r, pl.ds(3 * L, L)]
            acc_i = u0 * a0 + u1 * a1 + u2 * a2 + u3 * a3
            acc_j = u0 * b0 + u1 * b1 + u2 * b2 + u3 * b3
            rvec = jnp.full((L,), r, _i32)
            plsc.store_scatter(oi_v, [rvec], plsc.cumsum(acc_i), mask=m15)
            plsc.store_scatter(oj_v, [rvec], plsc.cumsum(acc_j), mask=m15)

        pltpu.sync_copy(oi_v, oi_hbm.at[pl.ds(base, BPW)])
        pltpu.sync_copy(oj_v, oj_hbm.at[pl.ds(base, BPW)])

    return k(rk_u, rk_i, rk_j, gu, gi)


def _sorted_stream(idx, wpos):
    n = idx.shape[0]
    iota = lax.iota(_i32, n)
    packed = lax.bitwise_or(
        lax.shift_left(lax.shift_right_logical(idx, 7), wpos), iota)
    sp, carr = lax.sort([packed, lax.bitwise_and(idx, _i32(127))],
                        num_keys=1)
    pos = lax.bitwise_and(sp, _i32((1 << wpos) - 1))
    blocks = lax.shift_right_logical(sp, wpos)
    rank = lax.sort([pos, iota], num_keys=1)[1]
    isnew = jnp.concatenate(
        [jnp.ones((1,), _i32), (jnp.diff(blocks) != 0).astype(_i32)])
    slot = jnp.cumsum(isnew, dtype=_i32) - 1
    slotinfo = slot * 2 + isnew
    ckey = jnp.where(isnew == 1, slot, _i32(n))
    dcomp = lax.sort([ckey, blocks], num_keys=1)[1]
    dlist = jnp.concatenate([dcomp, jnp.zeros((16,), _i32)])
    return carr, slotinfo, dlist, rank


def kernel(user, item_i, item_j, embed_user, embed_item):
    user = user.astype(_i32)
    item_i = item_i.astype(_i32)
    item_j = item_j.astype(_i32)

    carr_u, slotinfo_u, dlist_u, rank_u = _sorted_stream(user, 14)
    gu_flat = _gather_kernel(carr_u, slotinfo_u, dlist_u, embed_user.T, B)

    items = jnp.concatenate([item_i, item_j])
    carr_i, slotinfo_i, dlist_i, rank_it = _sorted_stream(items, 15)
    gi_flat = _gather_kernel(carr_i, slotinfo_i, dlist_i, embed_item.T, 2 * B)

    gu = gu_flat.reshape(B, F)
    gi = gi_flat.reshape(2 * B, F)

    return _dot_kernel(rank_u, rank_it[:B], rank_it[B:], gu, gi)
